# baseline (speedup 1.0000x reference)
import functools

import jax
import jax.numpy as jnp
from jax import lax
from jax.experimental import pallas as pl
from jax.experimental.pallas import tpu as pltpu
from jax.experimental.pallas import tpu_sc as plsc

_B = 16384
_K = 20
_D = 64
_NC = 2
_NS = 16
_NW = _NC * _NS
_CB = _B // _NW
_C = 32
_NCHUNK = _CB // _C
_IDXCHUNK = 128
_TR = 500000


def _sc_body(cw_hbm, tw_hbm, ow_hbm, v_hbm, u_hbm, score_hbm, sumexp_hbm,
             idx_c, idx_t, idx_o, idx_c2, idx_t2, idx_o2,
             vrows, trows, orows, norm_buf, score_buf, sumexp_buf, sem):
    wid = lax.axis_index("s") * _NC + lax.axis_index("c")
    wbase = wid * _CB
    lanes = lax.iota(jnp.int32, 16)
    last = lanes == 15

    def chunk_body(i, _):
        base = wbase + i * _C
        pltpu.sync_copy(cw_hbm.at[pl.ds(base, _C)], idx_c.at[pl.ds(0, _C)])
        pltpu.sync_copy(tw_hbm.at[pl.ds(base, _C)], idx_t.at[pl.ds(0, _C)])
        pltpu.sync_copy(ow_hbm.at[pl.ds(base * _K, _C * _K)],
                        idx_o.at[pl.ds(0, _C * _K)])
        for j in range(_C // 16):
            idx_c2[pl.ds(j * 16, 16)] = idx_c[pl.ds(j * 16, 16)] >> 1
            idx_t2[pl.ds(j * 16, 16)] = idx_t[pl.ds(j * 16, 16)] >> 1
        for j in range(_C * _K // 16):
            idx_o2[pl.ds(j * 16, 16)] = idx_o[pl.ds(j * 16, 16)] >> 1
        copies = [
            pltpu.async_copy(v_hbm.at[idx_c2], vrows, sem),
            pltpu.async_copy(u_hbm.at[idx_t2], trows, sem),
        ]
        for g in range(_C * _K // _IDXCHUNK):
            copies.append(pltpu.async_copy(
                u_hbm.at[idx_o2.at[pl.ds(g * _IDXCHUNK, _IDXCHUNK)]],
                orows.at[pl.ds(g * _IDXCHUNK, _IDXCHUNK)], sem))
        for cp in copies:
            cp.wait()

        def ebody(e, _):
            cvec = idx_c[pl.ds(e, 16)]
            tvec = idx_t[pl.ds(e, 16)]
            ovec0 = (idx_o[pl.ds(e * _K, 16)] & 1) * _D
            ovec1 = (idx_o[pl.ds(e * _K + 16, 16)] & 1) * _D
            pc = (cvec[0] & 1) * _D
            pt = (tvec[0] & 1) * _D
            c = [vrows[e, pl.ds(pc + 16 * j, 16)] for j in range(4)]
            prod = c[0] * trows[e, pl.ds(pt, 16)]
            for j in range(1, 4):
                prod = prod + c[j] * trows[e, pl.ds(pt + 16 * j, 16)]
            plsc.store_scatter(score_buf,
                               [jnp.full((16,), i * _C + e, jnp.int32)],
                               plsc.cumsum(prod), mask=last)
            for k in range(_K):
                p = ovec0[k] if k < 16 else ovec1[k - 16]
                row = e * _K + k
                acc = c[0] * orows[row, pl.ds(p, 16)]
                for j in range(1, 4):
                    acc = acc + c[j] * orows[row, pl.ds(p + 16 * j, 16)]
                plsc.store_scatter(norm_buf,
                                   [jnp.full((16,), k * _C + e, jnp.int32)],
                                   plsc.cumsum(acc), mask=last)
            return 0

        lax.fori_loop(0, _C, ebody, 0)

        for g in range(_C // 16):
            s = jnp.exp(norm_buf[pl.ds(g * 16, 16)])
            for k in range(1, _K):
                s = s + jnp.exp(norm_buf[pl.ds(k * _C + g * 16, 16)])
            sumexp_buf[pl.ds(i * _C + g * 16, 16)] = s
        return 0

    lax.fori_loop(0, _NCHUNK, chunk_body, 0)

    pltpu.sync_copy(score_buf, score_hbm.at[pl.ds(wbase, _CB)])
    pltpu.sync_copy(sumexp_buf, sumexp_hbm.at[pl.ds(wbase, _CB)])


def _nll_body(score_ref, sumexp_ref, o_ref):
    s = score_ref[...]
    z = sumexp_ref[...]
    o_ref[0, 0] = (jnp.sum(jnp.log(z)) - jnp.sum(s)) / _B


def kernel(center_words, target_words, outer_words, V, U):
    cw = center_words.reshape(_B)
    tw = target_words.reshape(_B)
    ow = outer_words.reshape(_B * _K)
    v2 = V.reshape(_TR, 2 * _D)
    u2 = U.reshape(_TR, 2 * _D)

    mesh = plsc.VectorSubcoreMesh(core_axis_name="c", subcore_axis_name="s")
    sc = functools.partial(
        pl.kernel, mesh=mesh,
        compiler_params=pltpu.CompilerParams(
            use_tc_tiling_on_sc=True, needs_layout_passes=False),
        out_type=[jax.ShapeDtypeStruct((_B,), jnp.float32),
                  jax.ShapeDtypeStruct((_B,), jnp.float32)],
        scratch_types=[
            pltpu.VMEM((_C + 16,), jnp.int32),
            pltpu.VMEM((_C + 16,), jnp.int32),
            pltpu.VMEM((_C * _K + 16,), jnp.int32),
            pltpu.VMEM((_C,), jnp.int32),
            pltpu.VMEM((_C,), jnp.int32),
            pltpu.VMEM((_C * _K,), jnp.int32),
            pltpu.VMEM((_C, 2 * _D), jnp.float32),
            pltpu.VMEM((_C, 2 * _D), jnp.float32),
            pltpu.VMEM((_C * _K, 2 * _D), jnp.float32),
            pltpu.VMEM((_C * _K,), jnp.float32),
            pltpu.VMEM((_CB,), jnp.float32),
            pltpu.VMEM((_CB,), jnp.float32),
            pltpu.SemaphoreType.DMA,
        ],
    )(_sc_body)
    score, sumexp = sc(cw, tw, ow, v2, u2)

    out = pl.pallas_call(
        _nll_body,
        out_shape=jax.ShapeDtypeStruct((1, 1), jnp.float32),
        out_specs=pl.BlockSpec(memory_space=pltpu.SMEM),
    )(score.reshape(128, 128), sumexp.reshape(128, 128))
    return out[0, 0]


# speedup vs baseline: 1.1515x; 1.1515x over previous
import functools

import jax
import jax.numpy as jnp
from jax import lax
from jax.experimental import pallas as pl
from jax.experimental.pallas import tpu as pltpu
from jax.experimental.pallas import tpu_sc as plsc

_B = 16384
_K = 20
_D = 64
_NC = 2
_NS = 16
_NW = _NC * _NS
_CB = _B // _NW
_C = 32
_NCHUNK = _CB // _C
_IDXCHUNK = 128


def _sc_body(cw_hbm, tw_hbm, ow_hbm, v_hbm, u_hbm, score_hbm, sumexp_hbm,
             idx_c0, idx_t0, idx_o0, vrows0, trows0, orows0,
             idx_c1, idx_t1, idx_o1, vrows1, trows1, orows1,
             score_buf, sumexp_buf, sem0, sem1):
    wid = lax.axis_index("s") * _NC + lax.axis_index("c")
    wbase = wid * _CB
    lanes = lax.iota(jnp.int32, 16)
    bufs = ((idx_c0, idx_t0, idx_o0, vrows0, trows0, orows0, sem0),
            (idx_c1, idx_t1, idx_o1, vrows1, trows1, orows1, sem1))

    def stage(i, b):
        idx_c, idx_t, idx_o, vrows, trows, orows, sem = bufs[b]
        base = wbase + i * _C
        pltpu.sync_copy(cw_hbm.at[pl.ds(base, _C)], idx_c)
        pltpu.sync_copy(tw_hbm.at[pl.ds(base, _C)], idx_t)
        pltpu.sync_copy(ow_hbm.at[pl.ds(base * _K, _C * _K)], idx_o)
        copies = [
            pltpu.async_copy(v_hbm.at[idx_c], vrows, sem),
            pltpu.async_copy(u_hbm.at[idx_t], trows, sem),
        ]
        for g in range(_C * _K // _IDXCHUNK):
            copies.append(pltpu.async_copy(
                u_hbm.at[idx_o.at[pl.ds(g * _IDXCHUNK, _IDXCHUNK)]],
                orows.at[pl.ds(g * _IDXCHUNK, _IDXCHUNK)], sem))
        return copies

    def compute(i, b):
        _, _, _, vrows, trows, orows, _ = bufs[b]
        for g in range(_C // 16):
            rowe = g * 16 + lanes
            lk = rowe * _K
            zero = jnp.zeros((16,), jnp.float32)

            def dbody(d, carry):
                dd = (lanes + d) & (_D - 1)
                c_d = plsc.load_gather(vrows, [rowe, dd])
                t_d = plsc.load_gather(trows, [rowe, dd])
                acc_t = carry[0] + t_d * c_d
                accs = []
                for k in range(_K):
                    o_d = plsc.load_gather(orows, [lk + k, dd])
                    accs.append(carry[1 + k] + o_d * c_d)
                return (acc_t, *accs)

            out = lax.fori_loop(0, _D, dbody, (zero,) * (_K + 1))
            s = jnp.exp(out[1])
            for k in range(2, _K + 1):
                s = s + jnp.exp(out[k])
            off = i * _C + g * 16
            score_buf[pl.ds(off, 16)] = out[0]
            sumexp_buf[pl.ds(off, 16)] = s

    pending = stage(0, 0)
    for i in range(_NCHUNK):
        nxt = stage(i + 1, (i + 1) % 2) if i + 1 < _NCHUNK else None
        for cp in pending:
            cp.wait()
        compute(i, i % 2)
        pending = nxt

    pltpu.sync_copy(score_buf, score_hbm.at[pl.ds(wbase, _CB)])
    pltpu.sync_copy(sumexp_buf, sumexp_hbm.at[pl.ds(wbase, _CB)])


def _nll_body(score_ref, sumexp_ref, o_ref):
    s = score_ref[...]
    z = sumexp_ref[...]
    o_ref[0, 0] = (jnp.sum(jnp.log(z)) - jnp.sum(s)) / _B


def kernel(center_words, target_words, outer_words, V, U):
    cw = center_words.reshape(_B)
    tw = target_words.reshape(_B)
    ow = outer_words.reshape(_B * _K)

    mesh = plsc.VectorSubcoreMesh(core_axis_name="c", subcore_axis_name="s")
    buf_set = [
        pltpu.VMEM((_C,), jnp.int32),
        pltpu.VMEM((_C,), jnp.int32),
        pltpu.VMEM((_C * _K,), jnp.int32),
        pltpu.VMEM((_C, _D), jnp.float32),
        pltpu.VMEM((_C, _D), jnp.float32),
        pltpu.VMEM((_C * _K, _D), jnp.float32),
    ]
    sc = functools.partial(
        pl.kernel, mesh=mesh,
        compiler_params=pltpu.CompilerParams(
            use_tc_tiling_on_sc=False, needs_layout_passes=False),
        out_type=[jax.ShapeDtypeStruct((_B,), jnp.float32),
                  jax.ShapeDtypeStruct((_B,), jnp.float32)],
        scratch_types=buf_set + buf_set + [
            pltpu.VMEM((_CB,), jnp.float32),
            pltpu.VMEM((_CB,), jnp.float32),
            pltpu.SemaphoreType.DMA,
            pltpu.SemaphoreType.DMA,
        ],
    )(_sc_body)
    score, sumexp = sc(cw, tw, ow, V, U)

    out = pl.pallas_call(
        _nll_body,
        out_shape=jax.ShapeDtypeStruct((1, 1), jnp.float32),
        out_specs=pl.BlockSpec(memory_space=pltpu.SMEM),
    )(score.reshape(128, 128), sumexp.reshape(128, 128))
    return out[0, 0]


# speedup vs baseline: 1.5854x; 1.3769x over previous
import functools

import jax
import jax.numpy as jnp
from jax import lax
from jax.experimental import pallas as pl
from jax.experimental.pallas import tpu as pltpu
from jax.experimental.pallas import tpu_sc as plsc

_B = 16384
_K = 20
_D = 64
_NC = 2
_NS = 16
_NW = _NC * _NS
_CB = _B // _NW
_C = 16
_NCHUNK = _CB // _C
_IDXCHUNK = 128
_TR = 500000
_NSLAB = 7813
_RFULL = 244

_CPARAMS = pltpu.CompilerParams(
    use_tc_tiling_on_sc=True, needs_layout_passes=False)


def _transpose_slab(slab, tbuf, j0s, lanes):
    def dbody(d, _):
        dd = (lanes + d) & (_D - 1)
        for j0 in j0s:
            jv = j0 + lanes
            src = plsc.load_gather(slab, [dd, jv])
            plsc.store_scatter(tbuf, [jv >> 1, (jv & 1) * _D + dd], src)
        return 0

    lax.fori_loop(0, _D, dbody, 0)


def _zv_body(vt_hbm, vtail_hbm, out_hbm, slab0, tbuf0, slab1, tbuf1,
             semi0, semi1, semo0, semo1):
    wid = lax.axis_index("s") * _NC + lax.axis_index("c")
    lanes = lax.iota(jnp.int32, 16)
    j0s = tuple(range(0, 128, 16))
    bufs = ((slab0, tbuf0, semi0, semo0), (slab1, tbuf1, semi1, semo1))

    def stage(r, b):
        slab, _, semi, _ = bufs[b]
        s = r * _NW + wid
        pltpu.make_async_copy(
            vt_hbm.at[:, pl.ds(s * 128, 128)], slab, semi).start()

    def wait_in(b):
        slab, _, semi, _ = bufs[b]
        pltpu.make_async_copy(vt_hbm.at[:, pl.ds(0, 128)], slab, semi).wait()

    def wait_out(b):
        _, tbuf, _, semo = bufs[b]
        pltpu.make_async_copy(tbuf, out_hbm.at[pl.ds(0, 64)], semo).wait()

    def put(r, b):
        _, tbuf, _, semo = bufs[b]
        s = r * _NW + wid
        pltpu.make_async_copy(tbuf, out_hbm.at[pl.ds(s * 64, 64)], semo).start()

    stage(0, 0)

    def rbody(r, _):
        p = r % 2

        def go(b):
            slab, tbuf, _, _ = bufs[b]

            @pl.when(r + 1 < _RFULL)
            def _():
                stage(r + 1, 1 - b)

            wait_in(b)

            @pl.when(r >= 2)
            def _():
                wait_out(b)

            _transpose_slab(slab, tbuf, j0s, lanes)
            put(r, b)

        @pl.when(p == 0)
        def _():
            go(0)

        @pl.when(p == 1)
        def _():
            go(1)

        return 0

    lax.fori_loop(0, _RFULL, rbody, 0)
    wait_out(0)
    wait_out(1)

    s_extra = _RFULL * _NW + wid

    @pl.when(wid < 4)
    def _():
        pltpu.sync_copy(vt_hbm.at[:, pl.ds(s_extra * 128, 128)], slab0)
        _transpose_slab(slab0, tbuf0, j0s, lanes)
        pltpu.sync_copy(tbuf0, out_hbm.at[pl.ds(s_extra * 64, 64)])

    @pl.when(wid == 4)
    def _():
        pltpu.sync_copy(vtail_hbm, tbuf0.at[pl.ds(0, 32), :])
        pltpu.sync_copy(tbuf0.at[pl.ds(0, 32), :],
                        out_hbm.at[pl.ds((_NSLAB - 1) * 64, 32)])


def _sc_body(cw_hbm, tw_hbm, ow_hbm, v_hbm, u_hbm, score_hbm, sumexp_hbm,
             idx_c0, idx_t0, idx_o0, idx2_c0, idx2_t0, idx2_o0,
             vrows0, trows0, orows0,
             idx_c1, idx_t1, idx_o1, idx2_c1, idx2_t1, idx2_o1,
             vrows1, trows1, orows1,
             score_buf, sumexp_buf, sem0, sem1):
    wid = lax.axis_index("s") * _NC + lax.axis_index("c")
    wbase = wid * _CB
    lanes = lax.iota(jnp.int32, 16)
    bufs = ((idx_c0, idx_t0, idx_o0, idx2_c0, idx2_t0, idx2_o0,
             vrows0, trows0, orows0, sem0),
            (idx_c1, idx_t1, idx_o1, idx2_c1, idx2_t1, idx2_o1,
             vrows1, trows1, orows1, sem1))

    def stage(i, b):
        idx_c, idx_t, idx_o, idx2_c, idx2_t, idx2_o, vrows, trows, orows, \
            sem = bufs[b]
        base = wbase + i * _C
        pltpu.sync_copy(cw_hbm.at[pl.ds(base, _C)], idx_c)
        pltpu.sync_copy(tw_hbm.at[pl.ds(base, _C)], idx_t)
        pltpu.sync_copy(ow_hbm.at[pl.ds(base * _K, _C * _K)], idx_o)
        idx2_c[pl.ds(0, 16)] = idx_c[pl.ds(0, 16)] >> 1
        idx2_t[pl.ds(0, 16)] = idx_t[pl.ds(0, 16)] >> 1
        for j in range(_C * _K // 16):
            idx2_o[pl.ds(j * 16, 16)] = idx_o[pl.ds(j * 16, 16)] >> 1
        pltpu.async_copy(v_hbm.at[idx2_c], vrows, sem)
        pltpu.async_copy(u_hbm.at[idx2_t], trows, sem)
        for g in range((_C * _K + _IDXCHUNK - 1) // _IDXCHUNK):
            n = min(_IDXCHUNK, _C * _K - g * _IDXCHUNK)
            pltpu.async_copy(
                u_hbm.at[idx2_o.at[pl.ds(g * _IDXCHUNK, n)]],
                orows.at[pl.ds(g * _IDXCHUNK, n)], sem)

    def wait(b):
        _, _, _, _, _, _, vrows, trows, orows, sem = bufs[b]
        pltpu.make_async_copy(u_hbm.at[pl.ds(0, _C)], vrows, sem).wait()
        pltpu.make_async_copy(u_hbm.at[pl.ds(0, _C)], trows, sem).wait()
        pltpu.make_async_copy(u_hbm.at[pl.ds(0, _C * _K)], orows, sem).wait()

    def compute(i, b):
        idx_c, idx_t, idx_o, _, _, _, vrows, trows, orows, _ = bufs[b]
        pcv = (idx_c[pl.ds(0, 16)] & 1) * _D
        ptv = (idx_t[pl.ds(0, 16)] & 1) * _D
        pov = [(plsc.load_gather(idx_o, [lanes * _K + k]) & 1) * _D
               for k in range(_K)]
        lk = lanes * _K
        zero = jnp.zeros((16,), jnp.float32)

        def dbody(d, carry):
            dd = (lanes + d) & (_D - 1)
            c_d = plsc.load_gather(vrows, [lanes, pcv + dd])
            t_d = plsc.load_gather(trows, [lanes, ptv + dd])
            acc_t = carry[0] + t_d * c_d
            accs = []
            for k in range(_K):
                o_d = plsc.load_gather(orows, [lk + k, pov[k] + dd])
                accs.append(carry[1 + k] + o_d * c_d)
            return (acc_t, *accs)

        out = lax.fori_loop(0, _D, dbody, (zero,) * (_K + 1))
        s = jnp.exp(out[1])
        for k in range(2, _K + 1):
            s = s + jnp.exp(out[k])
        score_buf[pl.ds(i * _C, 16)] = out[0]
        sumexp_buf[pl.ds(i * _C, 16)] = s

    stage(0, 0)

    def chunk_body(i, _):
        p = i % 2

        def go(b):
            @pl.when(i + 1 < _NCHUNK)
            def _():
                stage(i + 1, 1 - b)

            wait(b)
            compute(i, b)

        @pl.when(p == 0)
        def _():
            go(0)

        @pl.when(p == 1)
        def _():
            go(1)

        return 0

    lax.fori_loop(0, _NCHUNK, chunk_body, 0)

    pltpu.sync_copy(score_buf, score_hbm.at[pl.ds(wbase, _CB)])
    pltpu.sync_copy(sumexp_buf, sumexp_hbm.at[pl.ds(wbase, _CB)])


def _nll_body(score_ref, sumexp_ref, o_ref):
    s = score_ref[...]
    z = sumexp_ref[...]
    o_ref[0, 0] = (jnp.sum(jnp.log(z)) - jnp.sum(s)) / _B


def kernel(center_words, target_words, outer_words, V, U):
    cw = center_words.reshape(_B)
    tw = target_words.reshape(_B)
    ow = outer_words.reshape(_B * _K)
    u2 = U.reshape(_TR, 2 * _D)

    mesh = plsc.VectorSubcoreMesh(core_axis_name="c", subcore_axis_name="s")
    zv = functools.partial(
        pl.kernel, mesh=mesh, compiler_params=_CPARAMS,
        out_type=jax.ShapeDtypeStruct((_TR, 2 * _D), jnp.float32),
        scratch_types=[
            pltpu.VMEM((_D, 128), jnp.float32),
            pltpu.VMEM((_D, 128), jnp.float32),
            pltpu.VMEM((_D, 128), jnp.float32),
            pltpu.VMEM((_D, 128), jnp.float32),
            pltpu.SemaphoreType.DMA,
            pltpu.SemaphoreType.DMA,
            pltpu.SemaphoreType.DMA,
            pltpu.SemaphoreType.DMA,
        ],
    )(_zv_body)
    v2 = zv(V.T, V[(_NSLAB - 1) * 128:].reshape(32, 2 * _D))

    buf_set = [
        pltpu.VMEM((_C,), jnp.int32),
        pltpu.VMEM((_C,), jnp.int32),
        pltpu.VMEM((_C * _K,), jnp.int32),
        pltpu.VMEM((_C,), jnp.int32),
        pltpu.VMEM((_C,), jnp.int32),
        pltpu.VMEM((_C * _K,), jnp.int32),
        pltpu.VMEM((_C, 2 * _D), jnp.float32),
        pltpu.VMEM((_C, 2 * _D), jnp.float32),
        pltpu.VMEM((_C * _K, 2 * _D), jnp.float32),
    ]
    sc = functools.partial(
        pl.kernel, mesh=mesh, compiler_params=_CPARAMS,
        out_type=[jax.ShapeDtypeStruct((_B,), jnp.float32),
                  jax.ShapeDtypeStruct((_B,), jnp.float32)],
        scratch_types=buf_set + buf_set + [
            pltpu.VMEM((_CB,), jnp.float32),
            pltpu.VMEM((_CB,), jnp.float32),
            pltpu.SemaphoreType.DMA,
            pltpu.SemaphoreType.DMA,
        ],
    )(_sc_body)
    score, sumexp = sc(cw, tw, ow, v2, u2)

    out = pl.pallas_call(
        _nll_body,
        out_shape=jax.ShapeDtypeStruct((1, 1), jnp.float32),
        out_specs=pl.BlockSpec(memory_space=pltpu.SMEM),
    )(score.reshape(128, 128), sumexp.reshape(128, 128))
    return out[0, 0]


# speedup vs baseline: 1.6032x; 1.0112x over previous
import functools

import jax
import jax.numpy as jnp
from jax import lax
from jax.experimental import pallas as pl
from jax.experimental.pallas import tpu as pltpu
from jax.experimental.pallas import tpu_sc as plsc

_B = 16384
_K = 20
_D = 64
_NC = 2
_NS = 16
_NW = _NC * _NS
_CB = _B // _NW
_C = 16
_NCHUNK = _CB // _C
_IDXCHUNK = 128
_TR = 500000
_NSLAB = 7813
_RFULL = 244

_CPARAMS = pltpu.CompilerParams(
    use_tc_tiling_on_sc=True, needs_layout_passes=False)


def _transpose_slab(slab, tbuf, j0s, lanes):
    jvs = [j0 + lanes for j0 in j0s]
    jhs = [jv >> 1 for jv in jvs]
    jls = [(jv & 1) * _D for jv in jvs]

    def dbody(t, _):
        for u in range(2):
            dd = (lanes + (t * 2 + u)) & (_D - 1)
            for jv, jh, jl in zip(jvs, jhs, jls):
                src = plsc.load_gather(slab, [dd, jv])
                plsc.store_scatter(tbuf, [jh, jl + dd], src)
        return 0

    lax.fori_loop(0, _D // 2, dbody, 0)


def _zv_body(vt_hbm, vtail_hbm, out_hbm, slab0, tbuf0, slab1, tbuf1,
             semi0, semi1, semo0, semo1):
    wid = lax.axis_index("s") * _NC + lax.axis_index("c")
    lanes = lax.iota(jnp.int32, 16)
    j0s = tuple(range(0, 128, 16))
    bufs = ((slab0, tbuf0, semi0, semo0), (slab1, tbuf1, semi1, semo1))

    def stage(r, b):
        slab, _, semi, _ = bufs[b]
        s = r * _NW + wid
        pltpu.make_async_copy(
            vt_hbm.at[:, pl.ds(s * 128, 128)], slab, semi).start()

    def wait_in(b):
        slab, _, semi, _ = bufs[b]
        pltpu.make_async_copy(vt_hbm.at[:, pl.ds(0, 128)], slab, semi).wait()

    def wait_out(b):
        _, tbuf, _, semo = bufs[b]
        pltpu.make_async_copy(tbuf, out_hbm.at[pl.ds(0, 64)], semo).wait()

    def put(r, b):
        _, tbuf, _, semo = bufs[b]
        s = r * _NW + wid
        pltpu.make_async_copy(tbuf, out_hbm.at[pl.ds(s * 64, 64)], semo).start()

    stage(0, 0)

    def rbody(r, _):
        p = r % 2

        def go(b):
            slab, tbuf, _, _ = bufs[b]

            @pl.when(r + 1 < _RFULL)
            def _():
                stage(r + 1, 1 - b)

            wait_in(b)

            @pl.when(r >= 2)
            def _():
                wait_out(b)

            _transpose_slab(slab, tbuf, j0s, lanes)
            put(r, b)

        @pl.when(p == 0)
        def _():
            go(0)

        @pl.when(p == 1)
        def _():
            go(1)

        return 0

    lax.fori_loop(0, _RFULL, rbody, 0)
    wait_out(0)
    wait_out(1)

    s_extra = _RFULL * _NW + wid

    @pl.when(wid < 4)
    def _():
        pltpu.sync_copy(vt_hbm.at[:, pl.ds(s_extra * 128, 128)], slab0)
        _transpose_slab(slab0, tbuf0, j0s, lanes)
        pltpu.sync_copy(tbuf0, out_hbm.at[pl.ds(s_extra * 64, 64)])

    @pl.when(wid == 4)
    def _():
        pltpu.sync_copy(vtail_hbm, tbuf0.at[pl.ds(0, 32), :])
        pltpu.sync_copy(tbuf0.at[pl.ds(0, 32), :],
                        out_hbm.at[pl.ds((_NSLAB - 1) * 64, 32)])


def _sc_body(cw_hbm, tw_hbm, ow_hbm, v_hbm, u_hbm, score_hbm, sumexp_hbm,
             idx_c0, idx_t0, idx_o0, idx2_c0, idx2_t0, idx2_o0,
             vrows0, trows0, orows0,
             idx_c1, idx_t1, idx_o1, idx2_c1, idx2_t1, idx2_o1,
             vrows1, trows1, orows1,
             score_buf, sumexp_buf, sem0, sem1):
    wid = lax.axis_index("s") * _NC + lax.axis_index("c")
    wbase = wid * _CB
    lanes = lax.iota(jnp.int32, 16)
    bufs = ((idx_c0, idx_t0, idx_o0, idx2_c0, idx2_t0, idx2_o0,
             vrows0, trows0, orows0, sem0),
            (idx_c1, idx_t1, idx_o1, idx2_c1, idx2_t1, idx2_o1,
             vrows1, trows1, orows1, sem1))

    def stage(i, b):
        idx_c, idx_t, idx_o, idx2_c, idx2_t, idx2_o, vrows, trows, orows, \
            sem = bufs[b]
        base = wbase + i * _C
        pltpu.sync_copy(cw_hbm.at[pl.ds(base, _C)], idx_c)
        pltpu.sync_copy(tw_hbm.at[pl.ds(base, _C)], idx_t)
        pltpu.sync_copy(ow_hbm.at[pl.ds(base * _K, _C * _K)], idx_o)
        idx2_c[pl.ds(0, 16)] = idx_c[pl.ds(0, 16)] >> 1
        idx2_t[pl.ds(0, 16)] = idx_t[pl.ds(0, 16)] >> 1
        for j in range(_C * _K // 16):
            idx2_o[pl.ds(j * 16, 16)] = idx_o[pl.ds(j * 16, 16)] >> 1
        pltpu.async_copy(v_hbm.at[idx2_c], vrows, sem)
        pltpu.async_copy(u_hbm.at[idx2_t], trows, sem)
        for g in range((_C * _K + _IDXCHUNK - 1) // _IDXCHUNK):
            n = min(_IDXCHUNK, _C * _K - g * _IDXCHUNK)
            pltpu.async_copy(
                u_hbm.at[idx2_o.at[pl.ds(g * _IDXCHUNK, n)]],
                orows.at[pl.ds(g * _IDXCHUNK, n)], sem)

    def wait(b):
        _, _, _, _, _, _, vrows, trows, orows, sem = bufs[b]
        pltpu.make_async_copy(u_hbm.at[pl.ds(0, _C)], vrows, sem).wait()
        pltpu.make_async_copy(u_hbm.at[pl.ds(0, _C)], trows, sem).wait()
        pltpu.make_async_copy(u_hbm.at[pl.ds(0, _C * _K)], orows, sem).wait()

    def compute(i, b):
        idx_c, idx_t, idx_o, _, _, _, vrows, trows, orows, _ = bufs[b]
        pcv = (idx_c[pl.ds(0, 16)] & 1) * _D
        ptv = (idx_t[pl.ds(0, 16)] & 1) * _D
        pov = [(plsc.load_gather(idx_o, [lanes * _K + k]) & 1) * _D
               for k in range(_K)]
        lk = lanes * _K
        zero = jnp.zeros((16,), jnp.float32)

        def dbody(d, carry):
            dd = (lanes + d) & (_D - 1)
            c_d = plsc.load_gather(vrows, [lanes, pcv + dd])
            t_d = plsc.load_gather(trows, [lanes, ptv + dd])
            acc_t = carry[0] + t_d * c_d
            accs = []
            for k in range(_K):
                o_d = plsc.load_gather(orows, [lk + k, pov[k] + dd])
                accs.append(carry[1 + k] + o_d * c_d)
            return (acc_t, *accs)

        out = lax.fori_loop(0, _D, dbody, (zero,) * (_K + 1))
        s = jnp.exp(out[1])
        for k in range(2, _K + 1):
            s = s + jnp.exp(out[k])
        score_buf[pl.ds(i * _C, 16)] = out[0]
        sumexp_buf[pl.ds(i * _C, 16)] = s

    stage(0, 0)

    def chunk_body(i, _):
        p = i % 2

        def go(b):
            @pl.when(i + 1 < _NCHUNK)
            def _():
                stage(i + 1, 1 - b)

            wait(b)
            compute(i, b)

        @pl.when(p == 0)
        def _():
            go(0)

        @pl.when(p == 1)
        def _():
            go(1)

        return 0

    lax.fori_loop(0, _NCHUNK, chunk_body, 0)

    pltpu.sync_copy(score_buf, score_hbm.at[pl.ds(wbase, _CB)])
    pltpu.sync_copy(sumexp_buf, sumexp_hbm.at[pl.ds(wbase, _CB)])


def _nll_body(score_ref, sumexp_ref, o_ref):
    s = score_ref[...]
    z = sumexp_ref[...]
    o_ref[0, 0] = (jnp.sum(jnp.log(z)) - jnp.sum(s)) / _B


def kernel(center_words, target_words, outer_words, V, U):
    cw = center_words.reshape(_B)
    tw = target_words.reshape(_B)
    ow = outer_words.reshape(_B * _K)
    u2 = U.reshape(_TR, 2 * _D)

    mesh = plsc.VectorSubcoreMesh(core_axis_name="c", subcore_axis_name="s")
    zv = functools.partial(
        pl.kernel, mesh=mesh, compiler_params=_CPARAMS,
        out_type=jax.ShapeDtypeStruct((_TR, 2 * _D), jnp.float32),
        scratch_types=[
            pltpu.VMEM((_D, 128), jnp.float32),
            pltpu.VMEM((_D, 128), jnp.float32),
            pltpu.VMEM((_D, 128), jnp.float32),
            pltpu.VMEM((_D, 128), jnp.float32),
            pltpu.SemaphoreType.DMA,
            pltpu.SemaphoreType.DMA,
            pltpu.SemaphoreType.DMA,
            pltpu.SemaphoreType.DMA,
        ],
    )(_zv_body)
    v2 = zv(V.T, V[(_NSLAB - 1) * 128:].reshape(32, 2 * _D))

    buf_set = [
        pltpu.VMEM((_C,), jnp.int32),
        pltpu.VMEM((_C,), jnp.int32),
        pltpu.VMEM((_C * _K,), jnp.int32),
        pltpu.VMEM((_C,), jnp.int32),
        pltpu.VMEM((_C,), jnp.int32),
        pltpu.VMEM((_C * _K,), jnp.int32),
        pltpu.VMEM((_C, 2 * _D), jnp.float32),
        pltpu.VMEM((_C, 2 * _D), jnp.float32),
        pltpu.VMEM((_C * _K, 2 * _D), jnp.float32),
    ]
    sc = functools.partial(
        pl.kernel, mesh=mesh, compiler_params=_CPARAMS,
        out_type=[jax.ShapeDtypeStruct((_B,), jnp.float32),
                  jax.ShapeDtypeStruct((_B,), jnp.float32)],
        scratch_types=buf_set + buf_set + [
            pltpu.VMEM((_CB,), jnp.float32),
            pltpu.VMEM((_CB,), jnp.float32),
            pltpu.SemaphoreType.DMA,
            pltpu.SemaphoreType.DMA,
        ],
    )(_sc_body)
    score, sumexp = sc(cw, tw, ow, v2, u2)

    out = pl.pallas_call(
        _nll_body,
        out_shape=jax.ShapeDtypeStruct((1, 1), jnp.float32),
        out_specs=pl.BlockSpec(memory_space=pltpu.SMEM),
    )(score.reshape(128, 128), sumexp.reshape(128, 128))
    return out[0, 0]


# speedup vs baseline: 1.6742x; 1.0443x over previous
import functools

import jax
import jax.numpy as jnp
from jax import lax
from jax.experimental import pallas as pl
from jax.experimental.pallas import tpu as pltpu
from jax.experimental.pallas import tpu_sc as plsc

_B = 16384
_K = 20
_D = 64
_NC = 2
_NS = 16
_NW = _NC * _NS
_CB = _B // _NW
_C = 16
_NCHUNK = _CB // _C
_IDXCHUNK = 128
_TR = 500000
_NSLAB = 7813
_RFULL = 244

_CPARAMS = pltpu.CompilerParams(
    use_tc_tiling_on_sc=True, needs_layout_passes=False)


def _transpose_slab(slab, tbuf, j0s, lanes):
    jvs = [j0 + lanes for j0 in j0s]
    jhs = [jv >> 1 for jv in jvs]
    jls = [(jv & 1) * _D for jv in jvs]

    def dbody(t, _):
        work = []
        for u in range(4):
            dd = (lanes + (t * 4 + u)) & (_D - 1)
            for jv, jh, jl in zip(jvs, jhs, jls):
                work.append((jh, jl + dd, plsc.load_gather(slab, [dd, jv])))
        for jh, col, src in work:
            plsc.store_scatter(tbuf, [jh, col], src)
        return 0

    lax.fori_loop(0, _D // 4, dbody, 0)


def _zv_body(vt_hbm, vtail_hbm, out_hbm, slab0, tbuf0, slab1, tbuf1,
             semi0, semi1, semo0, semo1):
    wid = lax.axis_index("s") * _NC + lax.axis_index("c")
    lanes = lax.iota(jnp.int32, 16)
    j0s = tuple(range(0, 128, 16))
    bufs = ((slab0, tbuf0, semi0, semo0), (slab1, tbuf1, semi1, semo1))

    def stage(r, b):
        slab, _, semi, _ = bufs[b]
        s = r * _NW + wid
        pltpu.make_async_copy(
            vt_hbm.at[:, pl.ds(s * 128, 128)], slab, semi).start()

    def wait_in(b):
        slab, _, semi, _ = bufs[b]
        pltpu.make_async_copy(vt_hbm.at[:, pl.ds(0, 128)], slab, semi).wait()

    def wait_out(b):
        _, tbuf, _, semo = bufs[b]
        pltpu.make_async_copy(tbuf, out_hbm.at[pl.ds(0, 64)], semo).wait()

    def put(r, b):
        _, tbuf, _, semo = bufs[b]
        s = r * _NW + wid
        pltpu.make_async_copy(tbuf, out_hbm.at[pl.ds(s * 64, 64)], semo).start()

    stage(0, 0)

    def rbody(r, _):
        p = r % 2

        def go(b):
            slab, tbuf, _, _ = bufs[b]

            @pl.when(r + 1 < _RFULL)
            def _():
                stage(r + 1, 1 - b)

            wait_in(b)

            @pl.when(r >= 2)
            def _():
                wait_out(b)

            _transpose_slab(slab, tbuf, j0s, lanes)
            put(r, b)

        @pl.when(p == 0)
        def _():
            go(0)

        @pl.when(p == 1)
        def _():
            go(1)

        return 0

    lax.fori_loop(0, _RFULL, rbody, 0)
    wait_out(0)
    wait_out(1)

    s_extra = _RFULL * _NW + wid

    @pl.when(wid < 4)
    def _():
        pltpu.sync_copy(vt_hbm.at[:, pl.ds(s_extra * 128, 128)], slab0)
        _transpose_slab(slab0, tbuf0, j0s, lanes)
        pltpu.sync_copy(tbuf0, out_hbm.at[pl.ds(s_extra * 64, 64)])

    @pl.when(wid == 4)
    def _():
        pltpu.sync_copy(vtail_hbm, tbuf0.at[pl.ds(0, 32), :])
        pltpu.sync_copy(tbuf0.at[pl.ds(0, 32), :],
                        out_hbm.at[pl.ds((_NSLAB - 1) * 64, 32)])


def _sc_body(cw_hbm, tw_hbm, ow_hbm, v_hbm, u_hbm, score_hbm, sumexp_hbm,
             idx_c0, idx_t0, idx_o0, idx2_c0, idx2_t0, idx2_o0,
             vrows0, trows0, orows0,
             idx_c1, idx_t1, idx_o1, idx2_c1, idx2_t1, idx2_o1,
             vrows1, trows1, orows1,
             score_buf, sumexp_buf, sem0, sem1):
    wid = lax.axis_index("s") * _NC + lax.axis_index("c")
    wbase = wid * _CB
    lanes = lax.iota(jnp.int32, 16)
    bufs = ((idx_c0, idx_t0, idx_o0, idx2_c0, idx2_t0, idx2_o0,
             vrows0, trows0, orows0, sem0),
            (idx_c1, idx_t1, idx_o1, idx2_c1, idx2_t1, idx2_o1,
             vrows1, trows1, orows1, sem1))

    def stage(i, b):
        idx_c, idx_t, idx_o, idx2_c, idx2_t, idx2_o, vrows, trows, orows, \
            sem = bufs[b]
        base = wbase + i * _C
        pltpu.sync_copy(cw_hbm.at[pl.ds(base, _C)], idx_c)
        pltpu.sync_copy(tw_hbm.at[pl.ds(base, _C)], idx_t)
        pltpu.sync_copy(ow_hbm.at[pl.ds(base * _K, _C * _K)], idx_o)
        idx2_c[pl.ds(0, 16)] = idx_c[pl.ds(0, 16)] >> 1
        idx2_t[pl.ds(0, 16)] = idx_t[pl.ds(0, 16)] >> 1
        for j in range(_C * _K // 16):
            idx2_o[pl.ds(j * 16, 16)] = idx_o[pl.ds(j * 16, 16)] >> 1
        pltpu.async_copy(v_hbm.at[idx2_c], vrows, sem)
        pltpu.async_copy(u_hbm.at[idx2_t], trows, sem)
        for g in range((_C * _K + _IDXCHUNK - 1) // _IDXCHUNK):
            n = min(_IDXCHUNK, _C * _K - g * _IDXCHUNK)
            pltpu.async_copy(
                u_hbm.at[idx2_o.at[pl.ds(g * _IDXCHUNK, n)]],
                orows.at[pl.ds(g * _IDXCHUNK, n)], sem)

    def wait(b):
        _, _, _, _, _, _, vrows, trows, orows, sem = bufs[b]
        pltpu.make_async_copy(u_hbm.at[pl.ds(0, _C)], vrows, sem).wait()
        pltpu.make_async_copy(u_hbm.at[pl.ds(0, _C)], trows, sem).wait()
        pltpu.make_async_copy(u_hbm.at[pl.ds(0, _C * _K)], orows, sem).wait()

    def compute(i, b):
        idx_c, idx_t, idx_o, _, _, _, vrows, trows, orows, _ = bufs[b]
        pcv = (idx_c[pl.ds(0, 16)] & 1) * _D
        ptv = (idx_t[pl.ds(0, 16)] & 1) * _D
        pov = [(plsc.load_gather(idx_o, [lanes * _K + k]) & 1) * _D
               for k in range(_K)]
        lk = lanes * _K
        zero = jnp.zeros((16,), jnp.float32)

        def dbody(d, carry):
            dd = (lanes + d) & (_D - 1)
            c_d = plsc.load_gather(vrows, [lanes, pcv + dd])
            t_d = plsc.load_gather(trows, [lanes, ptv + dd])
            acc_t = carry[0] + t_d * c_d
            accs = []
            for k in range(_K):
                o_d = plsc.load_gather(orows, [lk + k, pov[k] + dd])
                accs.append(carry[1 + k] + o_d * c_d)
            return (acc_t, *accs)

        out = lax.fori_loop(0, _D, dbody, (zero,) * (_K + 1))
        s = jnp.exp(out[1])
        for k in range(2, _K + 1):
            s = s + jnp.exp(out[k])
        score_buf[pl.ds(i * _C, 16)] = out[0]
        sumexp_buf[pl.ds(i * _C, 16)] = s

    stage(0, 0)

    def chunk_body(i, _):
        p = i % 2

        def go(b):
            @pl.when(i + 1 < _NCHUNK)
            def _():
                stage(i + 1, 1 - b)

            wait(b)
            compute(i, b)

        @pl.when(p == 0)
        def _():
            go(0)

        @pl.when(p == 1)
        def _():
            go(1)

        return 0

    lax.fori_loop(0, _NCHUNK, chunk_body, 0)

    pltpu.sync_copy(score_buf, score_hbm.at[pl.ds(wbase, _CB)])
    pltpu.sync_copy(sumexp_buf, sumexp_hbm.at[pl.ds(wbase, _CB)])


def _nll_body(score_ref, sumexp_ref, o_ref):
    s = score_ref[...]
    z = sumexp_ref[...]
    o_ref[0, 0] = (jnp.sum(jnp.log(z)) - jnp.sum(s)) / _B


def kernel(center_words, target_words, outer_words, V, U):
    cw = center_words.reshape(_B)
    tw = target_words.reshape(_B)
    ow = outer_words.reshape(_B * _K)
    u2 = U.reshape(_TR, 2 * _D)

    mesh = plsc.VectorSubcoreMesh(core_axis_name="c", subcore_axis_name="s")
    zv = functools.partial(
        pl.kernel, mesh=mesh, compiler_params=_CPARAMS,
        out_type=jax.ShapeDtypeStruct((_TR, 2 * _D), jnp.float32),
        scratch_types=[
            pltpu.VMEM((_D, 128), jnp.float32),
            pltpu.VMEM((_D, 128), jnp.float32),
            pltpu.VMEM((_D, 128), jnp.float32),
            pltpu.VMEM((_D, 128), jnp.float32),
            pltpu.SemaphoreType.DMA,
            pltpu.SemaphoreType.DMA,
            pltpu.SemaphoreType.DMA,
            pltpu.SemaphoreType.DMA,
        ],
    )(_zv_body)
    v2 = zv(V.T, V[(_NSLAB - 1) * 128:].reshape(32, 2 * _D))

    buf_set = [
        pltpu.VMEM((_C,), jnp.int32),
        pltpu.VMEM((_C,), jnp.int32),
        pltpu.VMEM((_C * _K,), jnp.int32),
        pltpu.VMEM((_C,), jnp.int32),
        pltpu.VMEM((_C,), jnp.int32),
        pltpu.VMEM((_C * _K,), jnp.int32),
        pltpu.VMEM((_C, 2 * _D), jnp.float32),
        pltpu.VMEM((_C, 2 * _D), jnp.float32),
        pltpu.VMEM((_C * _K, 2 * _D), jnp.float32),
    ]
    sc = functools.partial(
        pl.kernel, mesh=mesh, compiler_params=_CPARAMS,
        out_type=[jax.ShapeDtypeStruct((_B,), jnp.float32),
                  jax.ShapeDtypeStruct((_B,), jnp.float32)],
        scratch_types=buf_set + buf_set + [
            pltpu.VMEM((_CB,), jnp.float32),
            pltpu.VMEM((_CB,), jnp.float32),
            pltpu.SemaphoreType.DMA,
            pltpu.SemaphoreType.DMA,
        ],
    )(_sc_body)
    score, sumexp = sc(cw, tw, ow, v2, u2)

    out = pl.pallas_call(
        _nll_body,
        out_shape=jax.ShapeDtypeStruct((1, 1), jnp.float32),
        out_specs=pl.BlockSpec(memory_space=pltpu.SMEM),
    )(score.reshape(128, 128), sumexp.reshape(128, 128))
    return out[0, 0]
